# baseline (speedup 1.0000x reference)
import functools

import jax
import jax.numpy as jnp
from jax import lax
from jax.experimental import pallas as pl
from jax.experimental.pallas import tpu as pltpu
from jax.experimental.pallas import tpu_sc as plsc

_NC = 2
_NS = 16
_NW = _NC * _NS
_CH = 128
_G = 8
_LC = 10


def _sc_gather(table, idx_list, n_chunks, D):
    ntot = sum(n_chunks)
    rpc = _CH
    dt = table.dtype
    mesh = plsc.VectorSubcoreMesh(core_axis_name="c", subcore_axis_name="s")

    @functools.partial(
        pl.kernel,
        out_type=tuple(
            jax.ShapeDtypeStruct((nch * _NW * rpc, D), dt)
            for nch in n_chunks
        ),
        mesh=mesh,
        compiler_params=pltpu.CompilerParams(use_tc_tiling_on_sc=False),
        scratch_types=[
            pltpu.VMEM((ntot, _CH), jnp.int32),
            pltpu.VMEM((2, _G, _CH, D), dt),
            pltpu.SemaphoreType.DMA,
            pltpu.SemaphoreType.DMA,
        ],
    )
    def k(table_h, *refs):
        nin = len(idx_list)
        idx_hs = refs[:nin]
        out_hs = refs[nin:2 * nin]
        idx_v, rows_v, gsem, ssem = refs[2 * nin:]
        wid = lax.axis_index("s") * _NC + lax.axis_index("c")

        base = 0
        for idx_h, nch in zip(idx_hs, n_chunks):
            pltpu.sync_copy(
                idx_h.at[pl.ds(wid * nch, nch)], idx_v.at[pl.ds(base, nch)]
            )
            base += nch

        def drain_one_scatter(out_h):
            pltpu.make_async_copy(
                out_h.at[pl.ds(0, rpc)], rows_v.at[0, 0], ssem
            ).wait()

        def section(out_h, idx_base, out_base, nch):
            if nch % _G != 0 or nch // _G < 2:
                for j in range(nch):
                    pltpu.async_copy(
                        table_h.at[idx_v.at[idx_base + j]],
                        rows_v.at[0, 0], gsem,
                    ).wait()
                    pltpu.sync_copy(
                        rows_v.at[0, 0],
                        out_h.at[pl.ds(out_base + j * rpc, rpc)],
                    )
                return
            ng = nch // _G

            def body(g, carry):
                s = lax.rem(g, 2)

                @pl.when(g >= 2)
                def _():
                    for _ in range(_G):
                        drain_one_scatter(out_h)

                descs = []
                for b in range(_G):
                    descs.append(
                        pltpu.async_copy(
                            table_h.at[idx_v.at[idx_base + g * _G + b]],
                            rows_v.at[s, b],
                            gsem,
                        )
                    )
                for dsc in descs:
                    dsc.wait()
                for b in range(_G):
                    pltpu.async_copy(
                        rows_v.at[s, b],
                        out_h.at[pl.ds(out_base + (g * _G + b) * rpc, rpc)],
                        ssem,
                    )
                return carry

            lax.fori_loop(0, ng, body, 0)
            for _ in range(2 * _G):
                drain_one_scatter(out_h)

        idx_base = 0
        for out_h, nch in zip(out_hs, n_chunks):
            section(out_h, idx_base, wid * nch * rpc, nch)
            idx_base += nch

    return k(table, *idx_list)


def _tc_att(qt, keys4, mask3, w1t, b1c, w2t, b2c, B, L, D, units, Bb):
    nb = B // Bb
    QD = Bb // 4

    def body(q_ref, k_ref, m_ref, w1_ref, b1_ref, w2_ref, b2_ref, o_ref):
        l = pl.program_id(1)
        qv = q_ref[...]
        for j in range(_LC):
            k4t = k_ref[j].T
            kv = jnp.concatenate(
                [k4t[32 * m:32 * (m + 1), :] for m in range(4)], axis=1
            )
            x = jnp.concatenate([qv, kv, qv * kv], axis=0)
            h = jnp.dot(w1_ref[...], x, preferred_element_type=jnp.float32)
            h = jnp.maximum(h + b1_ref[...], 0.0)
            s = jnp.dot(w2_ref[...], h, preferred_element_type=jnp.float32)
            s = (s + b2_ref[...]) * m_ref[j]
            contrib = s * kv
            if j == 0:
                @pl.when(l == 0)
                def _():
                    o_ref[...] = contrib

                @pl.when(l > 0)
                def _():
                    o_ref[...] = o_ref[...] + contrib
            else:
                o_ref[...] = o_ref[...] + contrib

    return pl.pallas_call(
        body,
        grid=(nb, L // _LC),
        in_specs=[
            pl.BlockSpec((D, Bb), lambda i, l: (0, i)),
            pl.BlockSpec((_LC, QD, 128), lambda i, l: (l, i, 0)),
            pl.BlockSpec((_LC, 1, Bb), lambda i, l: (l, 0, i)),
            pl.BlockSpec((units, 3 * D), lambda i, l: (0, 0)),
            pl.BlockSpec((units, 1), lambda i, l: (0, 0)),
            pl.BlockSpec((1, units), lambda i, l: (0, 0)),
            pl.BlockSpec((1, 1), lambda i, l: (0, 0)),
        ],
        out_specs=pl.BlockSpec((D, Bb), lambda i, l: (0, i)),
        out_shape=jax.ShapeDtypeStruct((D, B), jnp.float32),
    )(qt, keys4, mask3, w1t, b1c, w2t, b2c)


def kernel(table, W1, b1, W2, b2, sparse_ids, seq_ids, target_id, mask):
    B, F = sparse_ids.shape
    L = seq_ids.shape[1]
    D = table.shape[1]
    units = W1.shape[1]
    Bb = 2048

    p_all = jnp.arange(B, dtype=jnp.int32)
    blk = p_all // Bb
    pin = p_all % Bb
    b_of_pos = blk * Bb + 4 * (pin % (Bb // 4)) + pin // (Bb // 4)
    pos_of_b = blk * Bb + (pin // 4) + (pin % 4) * (Bb // 4)

    sp = sparse_ids.astype(jnp.int32).reshape(-1, _CH)
    sq = seq_ids.astype(jnp.int32).T.reshape(-1, _CH)
    tg = target_id.astype(jnp.int32)[b_of_pos].reshape(-1, _CH)

    ns = (B * L) // (_NW * _CH)
    nt = B // (_NW * _CH)
    nf = (B * F) // (_NW * _CH)

    tbf = table.astype(jnp.bfloat16)
    keys_lb, query, field_rows = _sc_gather(
        tbf, [sq, tg, sp], [ns, nt, nf], D
    )
    keys128 = keys_lb.reshape(-1, 128)
    field128 = field_rows.reshape(-1, 128)

    qt = query.T
    mask3 = mask.T[:, b_of_pos].reshape(L, 1, B)
    keys4 = keys128.reshape(L, B // 4, 128)

    att_p = _tc_att(qt, keys4, mask3, W1.T.astype(jnp.bfloat16),
                    b1.reshape(units, 1),
                    W2.reshape(units, 1).T, b2.reshape(1, 1),
                    B=B, L=L, D=D, units=units, Bb=Bb)
    att = att_p[:, pos_of_b].T

    return jnp.concatenate(
        [field128.astype(jnp.float32).reshape(B, F * D), att], axis=1
    )


# speedup vs baseline: 1.5842x; 1.5842x over previous
import functools

import jax
import jax.numpy as jnp
from jax import lax
from jax.experimental import pallas as pl
from jax.experimental.pallas import tpu as pltpu
from jax.experimental.pallas import tpu_sc as plsc

_NC = 2
_NS = 16
_NW = _NC * _NS
_CH = 128
_G = 8
_LC = 10


def _sc_gather(table, idx_list, n_chunks, D):
    ntot = sum(n_chunks)
    rpc = _CH
    dt = table.dtype
    mesh = plsc.VectorSubcoreMesh(core_axis_name="c", subcore_axis_name="s")

    @functools.partial(
        pl.kernel,
        out_type=tuple(
            jax.ShapeDtypeStruct((nch * _NW * rpc, D), dt)
            for nch in n_chunks
        ),
        mesh=mesh,
        compiler_params=pltpu.CompilerParams(use_tc_tiling_on_sc=False),
        scratch_types=[
            pltpu.VMEM((ntot, _CH), jnp.int32),
            pltpu.VMEM((2, _G, _CH, D), dt),
            pltpu.SemaphoreType.DMA,
            pltpu.SemaphoreType.DMA,
        ],
    )
    def k(table_h, *refs):
        nin = len(idx_list)
        idx_hs = refs[:nin]
        out_hs = refs[nin:2 * nin]
        idx_v, rows_v, gsem, ssem = refs[2 * nin:]
        wid = lax.axis_index("s") * _NC + lax.axis_index("c")

        base = 0
        for idx_h, nch in zip(idx_hs, n_chunks):
            pltpu.sync_copy(
                idx_h.at[pl.ds(wid * nch, nch)], idx_v.at[pl.ds(base, nch)]
            )
            base += nch

        def drain_one_scatter(out_h):
            pltpu.make_async_copy(
                out_h.at[pl.ds(0, rpc)], rows_v.at[0, 0], ssem
            ).wait()

        def section(out_h, idx_base, out_base, nch):
            if nch % _G != 0 or nch // _G < 2:
                for j in range(nch):
                    pltpu.async_copy(
                        table_h.at[idx_v.at[idx_base + j]],
                        rows_v.at[0, 0], gsem,
                    ).wait()
                    pltpu.sync_copy(
                        rows_v.at[0, 0],
                        out_h.at[pl.ds(out_base + j * rpc, rpc)],
                    )
                return
            ng = nch // _G

            def body(g, carry):
                s = lax.rem(g, 2)

                @pl.when(g >= 2)
                def _():
                    for _ in range(_G):
                        drain_one_scatter(out_h)

                descs = []
                for b in range(_G):
                    descs.append(
                        pltpu.async_copy(
                            table_h.at[idx_v.at[idx_base + g * _G + b]],
                            rows_v.at[s, b],
                            gsem,
                        )
                    )
                for dsc in descs:
                    dsc.wait()
                for b in range(_G):
                    pltpu.async_copy(
                        rows_v.at[s, b],
                        out_h.at[pl.ds(out_base + (g * _G + b) * rpc, rpc)],
                        ssem,
                    )
                return carry

            lax.fori_loop(0, ng, body, 0)
            for _ in range(2 * _G):
                drain_one_scatter(out_h)

        idx_base = 0
        for out_h, nch in zip(out_hs, n_chunks):
            section(out_h, idx_base, wid * nch * rpc, nch)
            idx_base += nch

    return k(table, *idx_list)


def _tc_att(qt, keys4, mask3, w1t, b1c, w2t, b2c, B, L, D, units, Bb):
    nb = B // Bb
    QD = Bb // 4

    def body(q_ref, k_ref, m_ref, w1_ref, b1_ref, w2_ref, b2_ref, o_ref):
        l = pl.program_id(1)
        qv = q_ref[...]
        for j in range(_LC):
            k4t = k_ref[j].T
            kv = jnp.concatenate(
                [k4t[32 * m:32 * (m + 1), :] for m in range(4)], axis=1
            )
            x = jnp.concatenate([qv, kv, qv * kv], axis=0)
            h = jnp.dot(w1_ref[...], x, preferred_element_type=jnp.float32)
            h = jnp.maximum(h + b1_ref[...], 0.0)
            s = jnp.dot(w2_ref[...], h, preferred_element_type=jnp.float32)
            s = (s + b2_ref[...]) * m_ref[j]
            contrib = s * kv
            if j == 0:
                @pl.when(l == 0)
                def _():
                    o_ref[...] = contrib

                @pl.when(l > 0)
                def _():
                    o_ref[...] = o_ref[...] + contrib
            else:
                o_ref[...] = o_ref[...] + contrib

    return pl.pallas_call(
        body,
        grid=(nb, L // _LC),
        in_specs=[
            pl.BlockSpec((D, Bb), lambda i, l: (0, i)),
            pl.BlockSpec((_LC, QD, 128), lambda i, l: (l, i, 0)),
            pl.BlockSpec((_LC, 1, Bb), lambda i, l: (l, 0, i)),
            pl.BlockSpec((units, 3 * D), lambda i, l: (0, 0)),
            pl.BlockSpec((units, 1), lambda i, l: (0, 0)),
            pl.BlockSpec((1, units), lambda i, l: (0, 0)),
            pl.BlockSpec((1, 1), lambda i, l: (0, 0)),
        ],
        out_specs=pl.BlockSpec((D, Bb), lambda i, l: (0, i)),
        out_shape=jax.ShapeDtypeStruct((D, B), jnp.float32),
    )(qt, keys4, mask3, w1t, b1c, w2t, b2c)


def kernel(table, W1, b1, W2, b2, sparse_ids, seq_ids, target_id, mask):
    B, F = sparse_ids.shape
    L = seq_ids.shape[1]
    D = table.shape[1]
    units = W1.shape[1]
    Bb = 2048

    p_all = jnp.arange(B, dtype=jnp.int32)
    blk = p_all // Bb
    pin = p_all % Bb
    b_of_pos = blk * Bb + 4 * (pin % (Bb // 4)) + pin // (Bb // 4)
    pos_of_b = blk * Bb + (pin // 4) + (pin % 4) * (Bb // 4)

    sp = sparse_ids.astype(jnp.int32).reshape(-1, _CH)
    sq = seq_ids.astype(jnp.int32).T.reshape(-1, _CH)
    tg = target_id.astype(jnp.int32)[b_of_pos].reshape(-1, _CH)

    ns = (B * L) // (_NW * _CH)
    nt = B // (_NW * _CH)
    nf = (B * F) // (_NW * _CH)

    keys_lb, query, field_rows = _sc_gather(
        table, [sq, tg, sp], [ns, nt, nf], D
    )
    keys128 = keys_lb.reshape(-1, 128)
    field128 = field_rows.reshape(-1, 128)

    qt = query.T
    mask3 = mask.T[:, b_of_pos].reshape(L, 1, B)
    keys4 = keys128.reshape(L, B // 4, 128)

    att_p = _tc_att(qt, keys4, mask3, W1.T,
                    b1.reshape(units, 1),
                    W2.reshape(units, 1).T, b2.reshape(1, 1),
                    B=B, L=L, D=D, units=units, Bb=Bb)
    att = att_p[:, pos_of_b].T

    return jnp.concatenate([field128.reshape(B, F * D), att], axis=1)


# speedup vs baseline: 1.5984x; 1.0090x over previous
import functools

import jax
import jax.numpy as jnp
from jax import lax
from jax.experimental import pallas as pl
from jax.experimental.pallas import tpu as pltpu
from jax.experimental.pallas import tpu_sc as plsc

_NC = 2
_NS = 16
_NW = _NC * _NS
_CH = 128
_G = 8
_LC = 25


def _sc_gather(table, idx_list, n_chunks, D):
    ntot = sum(n_chunks)
    rpc = _CH
    dt = table.dtype
    mesh = plsc.VectorSubcoreMesh(core_axis_name="c", subcore_axis_name="s")

    @functools.partial(
        pl.kernel,
        out_type=tuple(
            jax.ShapeDtypeStruct((nch * _NW * rpc, D), dt)
            for nch in n_chunks
        ),
        mesh=mesh,
        compiler_params=pltpu.CompilerParams(use_tc_tiling_on_sc=False),
        scratch_types=[
            pltpu.VMEM((ntot, _CH), jnp.int32),
            pltpu.VMEM((2, _G, _CH, D), dt),
            pltpu.SemaphoreType.DMA,
            pltpu.SemaphoreType.DMA,
        ],
    )
    def k(table_h, *refs):
        nin = len(idx_list)
        idx_hs = refs[:nin]
        out_hs = refs[nin:2 * nin]
        idx_v, rows_v, gsem, ssem = refs[2 * nin:]
        wid = lax.axis_index("s") * _NC + lax.axis_index("c")

        base = 0
        for idx_h, nch in zip(idx_hs, n_chunks):
            pltpu.sync_copy(
                idx_h.at[pl.ds(wid * nch, nch)], idx_v.at[pl.ds(base, nch)]
            )
            base += nch

        def drain_one_scatter(out_h):
            pltpu.make_async_copy(
                out_h.at[pl.ds(0, rpc)], rows_v.at[0, 0], ssem
            ).wait()

        def section(out_h, idx_base, out_base, nch):
            if nch % _G != 0 or nch // _G < 2:
                for j in range(nch):
                    pltpu.async_copy(
                        table_h.at[idx_v.at[idx_base + j]],
                        rows_v.at[0, 0], gsem,
                    ).wait()
                    pltpu.sync_copy(
                        rows_v.at[0, 0],
                        out_h.at[pl.ds(out_base + j * rpc, rpc)],
                    )
                return
            ng = nch // _G

            def body(g, carry):
                s = lax.rem(g, 2)

                @pl.when(g >= 2)
                def _():
                    for _ in range(_G):
                        drain_one_scatter(out_h)

                descs = []
                for b in range(_G):
                    descs.append(
                        pltpu.async_copy(
                            table_h.at[idx_v.at[idx_base + g * _G + b]],
                            rows_v.at[s, b],
                            gsem,
                        )
                    )
                for dsc in descs:
                    dsc.wait()
                for b in range(_G):
                    pltpu.async_copy(
                        rows_v.at[s, b],
                        out_h.at[pl.ds(out_base + (g * _G + b) * rpc, rpc)],
                        ssem,
                    )
                return carry

            lax.fori_loop(0, ng, body, 0)
            for _ in range(2 * _G):
                drain_one_scatter(out_h)

        idx_base = 0
        for out_h, nch in zip(out_hs, n_chunks):
            section(out_h, idx_base, wid * nch * rpc, nch)
            idx_base += nch

    return k(table, *idx_list)


def _tc_att(qt, keys4, mask3, w1t, b1c, w2t, b2c, B, L, D, units, Bb):
    nb = B // Bb
    QD = Bb // 4

    def body(q_ref, k_ref, m_ref, w1_ref, b1_ref, w2_ref, b2_ref, o_ref):
        l = pl.program_id(1)
        qv = q_ref[...]
        for j in range(_LC):
            k4t = k_ref[j].T
            kv = jnp.concatenate(
                [k4t[32 * m:32 * (m + 1), :] for m in range(4)], axis=1
            )
            x = jnp.concatenate([qv, kv, qv * kv], axis=0)
            h = jnp.dot(w1_ref[...], x, preferred_element_type=jnp.float32)
            h = jnp.maximum(h + b1_ref[...], 0.0)
            s = jnp.dot(w2_ref[...], h, preferred_element_type=jnp.float32)
            s = (s + b2_ref[...]) * m_ref[j]
            contrib = s * kv
            if j == 0:
                @pl.when(l == 0)
                def _():
                    o_ref[...] = contrib

                @pl.when(l > 0)
                def _():
                    o_ref[...] = o_ref[...] + contrib
            else:
                o_ref[...] = o_ref[...] + contrib

    return pl.pallas_call(
        body,
        grid=(nb, L // _LC),
        in_specs=[
            pl.BlockSpec((D, Bb), lambda i, l: (0, i)),
            pl.BlockSpec((_LC, QD, 128), lambda i, l: (l, i, 0)),
            pl.BlockSpec((_LC, 1, Bb), lambda i, l: (l, 0, i)),
            pl.BlockSpec((units, 3 * D), lambda i, l: (0, 0)),
            pl.BlockSpec((units, 1), lambda i, l: (0, 0)),
            pl.BlockSpec((1, units), lambda i, l: (0, 0)),
            pl.BlockSpec((1, 1), lambda i, l: (0, 0)),
        ],
        out_specs=pl.BlockSpec((D, Bb), lambda i, l: (0, i)),
        out_shape=jax.ShapeDtypeStruct((D, B), jnp.float32),
    )(qt, keys4, mask3, w1t, b1c, w2t, b2c)


def kernel(table, W1, b1, W2, b2, sparse_ids, seq_ids, target_id, mask):
    B, F = sparse_ids.shape
    L = seq_ids.shape[1]
    D = table.shape[1]
    units = W1.shape[1]
    Bb = 2048

    p_all = jnp.arange(B, dtype=jnp.int32)
    blk = p_all // Bb
    pin = p_all % Bb
    b_of_pos = blk * Bb + 4 * (pin % (Bb // 4)) + pin // (Bb // 4)
    pos_of_b = blk * Bb + (pin // 4) + (pin % 4) * (Bb // 4)

    sp = sparse_ids.astype(jnp.int32).reshape(-1, _CH)
    sq = seq_ids.astype(jnp.int32).T.reshape(-1, _CH)
    tg = target_id.astype(jnp.int32)[b_of_pos].reshape(-1, _CH)

    ns = (B * L) // (_NW * _CH)
    nt = B // (_NW * _CH)
    nf = (B * F) // (_NW * _CH)

    keys_lb, query, field_rows = _sc_gather(
        table, [sq, tg, sp], [ns, nt, nf], D
    )
    keys128 = keys_lb.reshape(-1, 128)
    field128 = field_rows.reshape(-1, 128)

    qt = query.T
    mask3 = mask.T[:, b_of_pos].reshape(L, 1, B)
    keys4 = keys128.reshape(L, B // 4, 128)

    att_p = _tc_att(qt, keys4, mask3, W1.T,
                    b1.reshape(units, 1),
                    W2.reshape(units, 1).T, b2.reshape(1, 1),
                    B=B, L=L, D=D, units=units, Bb=Bb)
    att = att_p[:, pos_of_b].T

    return jnp.concatenate([field128.reshape(B, F * D), att], axis=1)
